# Initial kernel scaffold; baseline (speedup 1.0000x reference)
import jax
import jax.numpy as jnp
from jax.experimental import pallas as pl


def kernel(x_product, x_customer, edge_index_pp, edge_index_rev, edge_label_index, edge_label, i1_Wl, i1_bl, i1_Wr, i1_br, i1_att, i1_b, i2_Wl, i2_bl, i2_Wr, i2_br, i2_att, i2_b, i_lW, i_lb, u1_Wl, u1_bl, u1_Wr, u1_br, u1_att, u1_b, u2_Wl, u2_bl, u2_Wr, u2_br, u2_We, u2_att, u2_b, u3_Wl, u3_bl, u3_Wr, u3_br, u3_We, u3_att, u3_b, u_lW, u_lb, d1_W, d1_b, d2_W, d2_b):
    raise NotImplementedError("write your pallas kernel here")


# speedup vs baseline: 1.0359x; 1.0359x over previous
import functools

import jax
import jax.numpy as jnp
from jax.experimental import pallas as pl


def _matmul_bias_kernel(x_ref, w_ref, b_ref, o_ref):
    o_ref[...] = jnp.dot(x_ref[...], w_ref[...],
                         preferred_element_type=jnp.float32) + b_ref[...]


def _matmul_bias(x, w, b, block=1000):
    m, k = x.shape
    n = w.shape[1]
    return pl.pallas_call(
        _matmul_bias_kernel,
        grid=(m // block,),
        in_specs=[
            pl.BlockSpec((block, k), lambda i: (i, 0)),
            pl.BlockSpec((k, n), lambda i: (0, 0)),
            pl.BlockSpec((n,), lambda i: (0,)),
        ],
        out_specs=pl.BlockSpec((block, n), lambda i: (i, 0)),
        out_shape=jax.ShapeDtypeStruct((m, n), jnp.float32),
    )(x, w, b)


def _seg_softmax(alpha, dst, n):
    amax = jax.ops.segment_max(alpha, dst, num_segments=n)
    amax = jax.lax.stop_gradient(jnp.where(jnp.isfinite(amax), amax, 0.0))
    e = jnp.exp(alpha - amax[dst])
    denom = jax.ops.segment_sum(e, dst, num_segments=n)
    return e / (denom[dst] + 1e-16)


def _gatv2(x_src, x_dst, src, dst, Wl, bl, Wr, br, att, bias,
           self_loops=False, edge_attr=None, We=None):
    n_dst = x_dst.shape[0]
    xl = _matmul_bias(x_src, Wl, bl)
    xr = _matmul_bias(x_dst, Wr, br)
    if self_loops:
        loop = jnp.arange(n_dst, dtype=src.dtype)
        src = jnp.concatenate([src, loop])
        dst = jnp.concatenate([dst, loop])
    e = xl[src] + xr[dst]
    if edge_attr is not None:
        e = e + edge_attr @ We
    e = jax.nn.leaky_relu(e, 0.2)
    alpha = jnp.sum(e * att, axis=-1)
    a = _seg_softmax(alpha, dst, n_dst)
    out = jax.ops.segment_sum(a[:, None] * xl[src], dst, num_segments=n_dst)
    return out + bias


def kernel(x_product, x_customer, edge_index_pp, edge_index_rev, edge_label_index, edge_label, i1_Wl, i1_bl, i1_Wr, i1_br, i1_att, i1_b, i2_Wl, i2_bl, i2_Wr, i2_br, i2_att, i2_b, i_lW, i_lb, u1_Wl, u1_bl, u1_Wr, u1_br, u1_att, u1_b, u2_Wl, u2_bl, u2_Wr, u2_br, u2_We, u2_att, u2_b, u3_Wl, u3_bl, u3_Wr, u3_br, u3_We, u3_att, u3_b, u_lW, u_lb, d1_W, d1_b, d2_W, d2_b):
    src = edge_index_pp[0]; dst = edge_index_pp[1]
    xp = x_product; xc = x_customer
    h = jax.nn.relu(_gatv2(xp, xp, src, dst, i1_Wl, i1_bl, i1_Wr, i1_br, i1_att, i1_b, self_loops=True))
    h = jax.nn.relu(_gatv2(h, h, src, dst, i2_Wl, i2_bl, i2_Wr, i2_br, i2_att, i2_b, self_loops=True))
    z_prod = _matmul_bias(h, i_lW, i_lb)
    ph = jax.nn.relu(_gatv2(xp, xp, src, dst, u1_Wl, u1_bl, u1_Wr, u1_br, u1_att, u1_b, self_loops=True))
    rs = edge_index_rev[0]; rd = edge_index_rev[1]
    ch = jax.nn.relu(_gatv2(xp, xc, rs, rd, u2_Wl, u2_bl, u2_Wr, u2_br, u2_att, u2_b, edge_attr=edge_label, We=u2_We))
    ch = jax.nn.relu(_gatv2(ph, ch, rs, rd, u3_Wl, u3_bl, u3_Wr, u3_br, u3_att, u3_b, edge_attr=edge_label, We=u3_We))
    z_cust = _matmul_bias(ch, u_lW, u_lb)
    row = edge_label_index[0]; col = edge_label_index[1]
    z = jnp.concatenate([z_cust[row], z_prod[col]], axis=-1)
    z = jax.nn.relu(z @ d1_W + d1_b)
    return z @ d2_W + d2_b
